# baseline (speedup 1.0000x reference)
import functools

import jax
import jax.numpy as jnp
from jax import lax
from jax.experimental import pallas as pl
from jax.experimental.pallas import tpu as pltpu
from jax.experimental.pallas import tpu_sc as plsc

N_NODES = 10000
D = 256
E = 160000
NC = 2
NS = 16
NW = NC * NS
L = 16
C = 64
NCHUNK = E // C
CPW = -(-NCHUNK // NW)


def _sc_sqdist(z, src, dst):
    mesh = plsc.VectorSubcoreMesh(core_axis_name="c", subcore_axis_name="s")

    @functools.partial(
        pl.kernel,
        out_type=jax.ShapeDtypeStruct((E,), jnp.float32),
        mesh=mesh,
        scratch_types=[
            pltpu.VMEM((C,), jnp.int32),
            pltpu.VMEM((C,), jnp.int32),
            pltpu.VMEM((C, D), jnp.float32),
            pltpu.VMEM((C, D), jnp.float32),
            pltpu.VMEM((C,), jnp.float32),
            pltpu.SemaphoreType.DMA,
            pltpu.SemaphoreType.DMA,
        ],
        compiler_params=pltpu.CompilerParams(
            use_tc_tiling_on_sc=False, needs_layout_passes=False),
    )
    def k(z_hbm, src_hbm, dst_hbm, out_hbm,
          sidx, didx, rows_s, rows_d, qc, sem_s, sem_d):
        wid = lax.axis_index("s") * NC + lax.axis_index("c")

        def chunk_body(ci, carry):
            g = ci * NW + wid

            @pl.when(g < NCHUNK)
            def _process():
                base = g * C
                pltpu.sync_copy(src_hbm.at[pl.ds(base, C)], sidx)
                pltpu.sync_copy(dst_hbm.at[pl.ds(base, C)], didx)
                cp_s = pltpu.async_copy(z_hbm.at[sidx], rows_s, sem_s)
                cp_d = pltpu.async_copy(z_hbm.at[didx], rows_d, sem_d)
                cp_s.wait()
                cp_d.wait()
                for eb in range(C // L):
                    row_ids = lax.iota(jnp.int32, L) + (eb * L)

                    def dim_body(kk, acc):
                        col = lax.broadcast(kk, (L,))
                        a = plsc.load_gather(rows_s, [row_ids, col])
                        b = plsc.load_gather(rows_d, [row_ids, col])
                        dlt = a - b + jnp.float32(1e-6)
                        return acc + dlt * dlt

                    acc = lax.fori_loop(0, D, dim_body,
                                        jnp.zeros((L,), jnp.float32))
                    qc[pl.ds(eb * L, L)] = acc
                pltpu.sync_copy(qc, out_hbm.at[pl.ds(base, C)])

            return carry

        lax.fori_loop(0, CPW, chunk_body, 0)

    return k(z, src, dst)


def _tc_finalize(q):
    rows = E // 128

    def body(q_ref, o_ref):
        qv = q_ref[...]
        d = lax.rsqrt(qv)
        m = jnp.max(d)
        o_ref[...] = jnp.exp(d - m)

    out = pl.pallas_call(
        body,
        out_shape=jax.ShapeDtypeStruct((rows, 128), jnp.float32),
    )(q.reshape(rows, 128))
    return out.reshape(E)


def kernel(z, edge_index, p):
    idx = edge_index.astype(jnp.int32)
    q = _sc_sqdist(z, idx[0], idx[1])
    return _tc_finalize(q)


# speedup vs baseline: 1.1688x; 1.1688x over previous
import functools

import jax
import jax.numpy as jnp
from jax import lax
from jax.experimental import pallas as pl
from jax.experimental.pallas import tpu as pltpu
from jax.experimental.pallas import tpu_sc as plsc

D = 256
E = 160000
NC = 2
NS = 16
NW = NC * NS
L = 16
EW = E // NW
C = 64
NCH = -(-EW // C)
LAST_BASE = EW - C


def _sc_sqdist(z, src, dst):
    mesh = plsc.VectorSubcoreMesh(core_axis_name="c", subcore_axis_name="s")

    @functools.partial(
        pl.kernel,
        out_type=jax.ShapeDtypeStruct((E,), jnp.float32),
        mesh=mesh,
        scratch_types=[
            pltpu.VMEM((EW,), jnp.int32),
            pltpu.VMEM((EW,), jnp.int32),
            pltpu.VMEM((C, D), jnp.float32),
            pltpu.VMEM((C, D), jnp.float32),
            pltpu.VMEM((C, D), jnp.float32),
            pltpu.VMEM((C, D), jnp.float32),
            pltpu.VMEM((EW,), jnp.float32),
            pltpu.SemaphoreType.DMA,
            pltpu.SemaphoreType.DMA,
            pltpu.SemaphoreType.DMA,
            pltpu.SemaphoreType.DMA,
        ],
        compiler_params=pltpu.CompilerParams(
            use_tc_tiling_on_sc=False, needs_layout_passes=False),
    )
    def k(z_hbm, src_hbm, dst_hbm, out_hbm,
          sidx, didx, sA, dA, sB, dB, qv, sem_sA, sem_dA, sem_sB, sem_dB):
        wid = lax.axis_index("s") * NC + lax.axis_index("c")
        ebase = wid * EW
        pltpu.sync_copy(src_hbm.at[pl.ds(ebase, EW)], sidx)
        pltpu.sync_copy(dst_hbm.at[pl.ds(ebase, EW)], didx)

        def chunk_base(c):
            return jnp.minimum(c * C, LAST_BASE)

        def issue(c, s_buf, d_buf, sem_s, sem_d):
            b = chunk_base(c)
            pltpu.async_copy(z_hbm.at[sidx.at[pl.ds(b, C)]], s_buf, sem_s)
            pltpu.async_copy(z_hbm.at[didx.at[pl.ds(b, C)]], d_buf, sem_d)

        def drain(s_buf, d_buf, sem_s, sem_d):
            pltpu.make_async_copy(z_hbm.at[sidx.at[pl.ds(0, C)]],
                                  s_buf, sem_s).wait()
            pltpu.make_async_copy(z_hbm.at[sidx.at[pl.ds(0, C)]],
                                  d_buf, sem_d).wait()

        zero16 = jnp.zeros((L,), jnp.int32)
        eps = jnp.float32(1e-6)

        def compute(c, s_buf, d_buf):
            qb = chunk_base(c)
            for eb in range(C // L):
                flat0 = (lax.iota(jnp.int32, L) + eb * L) * D

                def grp(_, carry):
                    acc, flat = carry
                    for _u in range(8):
                        a = plsc.load_gather(s_buf, [zero16, flat])
                        bb = plsc.load_gather(d_buf, [zero16, flat])
                        dlt = a - bb + eps
                        acc = acc + dlt * dlt
                        flat = flat + 1
                    return acc, flat

                acc, _ = lax.fori_loop(
                    0, D // 8, grp, (jnp.zeros((L,), jnp.float32), flat0))
                qv[pl.ds(qb + eb * L, L)] = acc

        issue(0, sA, dA, sem_sA, sem_dA)
        issue(1, sB, dB, sem_sB, sem_dB)

        def body(i2, carry):
            c0 = i2 * 2
            c1 = c0 + 1
            drain(sA, dA, sem_sA, sem_dA)
            compute(c0, sA, dA)
            issue(c0 + 2, sA, dA, sem_sA, sem_dA)
            drain(sB, dB, sem_sB, sem_dB)
            compute(c1, sB, dB)

            @pl.when(i2 < (NCH - 1) // 2 - 1)
            def _():
                issue(c1 + 2, sB, dB, sem_sB, sem_dB)

            return carry

        lax.fori_loop(0, (NCH - 1) // 2, body, 0)
        drain(sA, dA, sem_sA, sem_dA)
        compute(NCH - 1, sA, dA)
        pltpu.sync_copy(qv, out_hbm.at[pl.ds(ebase, EW)])

    return k(z, src, dst)


def _tc_finalize(q):
    rows = E // 128

    def body(q_ref, o_ref):
        qv = q_ref[...]
        d = lax.rsqrt(qv)
        m = jnp.max(d)
        o_ref[...] = jnp.exp(d - m)

    out = pl.pallas_call(
        body,
        out_shape=jax.ShapeDtypeStruct((rows, 128), jnp.float32),
    )(q.reshape(rows, 128))
    return out.reshape(E)


def kernel(z, edge_index, p):
    idx = edge_index.astype(jnp.int32)
    q = _sc_sqdist(z, idx[0], idx[1])
    return _tc_finalize(q)


# speedup vs baseline: 8.5070x; 7.2786x over previous
import functools

import jax
import jax.numpy as jnp
from jax import lax
from jax.experimental import pallas as pl
from jax.experimental.pallas import tpu as pltpu
from jax.experimental.pallas import tpu_sc as plsc

D = 256
E = 160000
NC = 2
NS = 16
NW = NC * NS
L = 16
EW = E // NW
C = 64
NCH = -(-EW // C)
LAST_BASE = EW - C


def _sc_sqdist(z, src, dst):
    mesh = plsc.VectorSubcoreMesh(core_axis_name="c", subcore_axis_name="s")

    @functools.partial(
        pl.kernel,
        out_type=jax.ShapeDtypeStruct((E,), jnp.float32),
        mesh=mesh,
        scratch_types=[
            pltpu.VMEM((EW,), jnp.int32),
            pltpu.VMEM((EW,), jnp.int32),
            pltpu.VMEM((C, D), jnp.float32),
            pltpu.VMEM((C, D), jnp.float32),
            pltpu.VMEM((C, D), jnp.float32),
            pltpu.VMEM((C, D), jnp.float32),
            pltpu.VMEM((EW,), jnp.float32),
            pltpu.SemaphoreType.DMA,
            pltpu.SemaphoreType.DMA,
            pltpu.SemaphoreType.DMA,
            pltpu.SemaphoreType.DMA,
        ],
        compiler_params=pltpu.CompilerParams(
            use_tc_tiling_on_sc=False, needs_layout_passes=False),
    )
    def k(z_hbm, src_hbm, dst_hbm, out_hbm,
          sidx, didx, sA, dA, sB, dB, qv, sem_sA, sem_dA, sem_sB, sem_dB):
        wid = lax.axis_index("s") * NC + lax.axis_index("c")
        ebase = wid * EW
        pltpu.sync_copy(src_hbm.at[pl.ds(ebase, EW)], sidx)
        pltpu.sync_copy(dst_hbm.at[pl.ds(ebase, EW)], didx)

        def chunk_base(c):
            return jnp.minimum(c * C, LAST_BASE)

        def issue(c, s_buf, d_buf, sem_s, sem_d):
            b = chunk_base(c)
            pltpu.async_copy(z_hbm.at[sidx.at[pl.ds(b, C)]], s_buf, sem_s)
            pltpu.async_copy(z_hbm.at[didx.at[pl.ds(b, C)]], d_buf, sem_d)

        def drain(s_buf, d_buf, sem_s, sem_d):
            pltpu.make_async_copy(z_hbm.at[sidx.at[pl.ds(0, C)]],
                                  s_buf, sem_s).wait()
            pltpu.make_async_copy(z_hbm.at[sidx.at[pl.ds(0, C)]],
                                  d_buf, sem_d).wait()

        zero16 = jnp.zeros((L,), jnp.int32)
        eps = jnp.float32(1e-6)
        rot0 = lax.iota(jnp.int32, L)

        def compute(c, s_buf, d_buf):
            qb = chunk_base(c)
            for eb in range(C // L):
                flat0 = (rot0 + eb * L) * D

                def grp(_, carry):
                    acc, flatbase = carry
                    for t in range(L):
                        colr = (rot0 + t) & (L - 1)
                        flat = flatbase + colr
                        a = plsc.load_gather(s_buf, [zero16, flat])
                        bb = plsc.load_gather(d_buf, [zero16, flat])
                        dlt = a - bb + eps
                        acc = acc + dlt * dlt
                    return acc, flatbase + L

                acc, _ = lax.fori_loop(
                    0, D // L, grp, (jnp.zeros((L,), jnp.float32), flat0))
                qv[pl.ds(qb + eb * L, L)] = acc

        issue(0, sA, dA, sem_sA, sem_dA)
        issue(1, sB, dB, sem_sB, sem_dB)

        def body(i2, carry):
            c0 = i2 * 2
            c1 = c0 + 1
            drain(sA, dA, sem_sA, sem_dA)
            compute(c0, sA, dA)
            issue(c0 + 2, sA, dA, sem_sA, sem_dA)
            drain(sB, dB, sem_sB, sem_dB)
            compute(c1, sB, dB)

            @pl.when(i2 < (NCH - 1) // 2 - 1)
            def _():
                issue(c1 + 2, sB, dB, sem_sB, sem_dB)

            return carry

        lax.fori_loop(0, (NCH - 1) // 2, body, 0)
        drain(sA, dA, sem_sA, sem_dA)
        compute(NCH - 1, sA, dA)
        pltpu.sync_copy(qv, out_hbm.at[pl.ds(ebase, EW)])

    return k(z, src, dst)


def _tc_finalize(q):
    rows = E // 128

    def body(q_ref, o_ref):
        qv = q_ref[...]
        d = lax.rsqrt(qv)
        m = jnp.max(d)
        o_ref[...] = jnp.exp(d - m)

    out = pl.pallas_call(
        body,
        out_shape=jax.ShapeDtypeStruct((rows, 128), jnp.float32),
    )(q.reshape(rows, 128))
    return out.reshape(E)


def kernel(z, edge_index, p):
    idx = edge_index.astype(jnp.int32)
    q = _sc_sqdist(z, idx[0], idx[1])
    return _tc_finalize(q)
